# Initial kernel scaffold; baseline (speedup 1.0000x reference)
import jax
import jax.numpy as jnp
from jax.experimental import pallas as pl


def kernel(edge_index, features, weight, bias):
    raise NotImplementedError("write your pallas kernel here")


# speedup vs baseline: 3.4265x; 3.4265x over previous
import functools

import jax
import jax.numpy as jnp
from jax import lax
from jax.experimental import pallas as pl
from jax.experimental.pallas import tpu as pltpu
from jax.experimental.pallas import tpu_sc as plsc

N_NODES = 10000
N_EDGES = 320000
F = 128

NC = 2
NS = 16
NW = NC * NS
CHUNK = 128
CPW = 80
E_PAD = NW * CPW * CHUNK
ACC_ROWS = 10240
RPT_Z = ACC_ROWS // NS
RPT_O = 624
TAIL_O = N_NODES - NS * RPT_O
ROW_BLK = N_NODES // 10


def _matmul_body(x_ref, w_ref, o_ref):
    o_ref[...] = jnp.dot(x_ref[...], w_ref[...],
                         preferred_element_type=jnp.float32)


def _matmul(x, w):
    return pl.pallas_call(
        _matmul_body,
        grid=(10,),
        in_specs=[
            pl.BlockSpec((ROW_BLK, F), lambda i: (i, 0)),
            pl.BlockSpec((F, F), lambda i: (0, 0)),
        ],
        out_specs=pl.BlockSpec((ROW_BLK, F), lambda i: (i, 0)),
        out_shape=jax.ShapeDtypeStruct((N_NODES, F), jnp.float32),
    )(x, w)


def _combine_body(p_ref, b_ref, o_ref):
    o_ref[...] = p_ref[0] + p_ref[1] + b_ref[...]


def _combine(p, b2d):
    return pl.pallas_call(
        _combine_body,
        grid=(10,),
        in_specs=[
            pl.BlockSpec((NC, ROW_BLK, F), lambda i: (0, i, 0)),
            pl.BlockSpec((1, F), lambda i: (0, 0)),
        ],
        out_specs=pl.BlockSpec((ROW_BLK, F), lambda i: (i, 0)),
        out_shape=jax.ShapeDtypeStruct((N_NODES, F), jnp.float32),
    )(p, b2d)


def _sc_scatter(h, src2d, dst2d, zeros):
    mesh = plsc.VectorSubcoreMesh(core_axis_name="c", subcore_axis_name="s")

    @functools.partial(
        pl.kernel,
        mesh=mesh,
        out_type=jax.ShapeDtypeStruct((NC, N_NODES, F), jnp.float32),
        scratch_types=[
            pltpu.VMEM((CPW, CHUNK), jnp.int32),
            pltpu.VMEM((CPW, CHUNK), jnp.int32),
            pltpu.VMEM((CHUNK, F), jnp.float32),
            pltpu.VMEM_SHARED((ACC_ROWS, F), jnp.float32),
            pltpu.SemaphoreType.DMA,
        ],
    )
    def k(h_hbm, src_hbm, dst_hbm, z_hbm, out_hbm, src_v, dst_v, rows_v, acc, sem):
        cid = lax.axis_index("c")
        sid = lax.axis_index("s")
        wid = sid * NC + cid

        pltpu.sync_copy(z_hbm.at[pl.ds(sid * RPT_Z, RPT_Z)],
                        acc.at[pl.ds(sid * RPT_Z, RPT_Z)])

        pltpu.sync_copy(src_hbm.at[pl.ds(wid * CPW, CPW)], src_v)
        pltpu.sync_copy(dst_hbm.at[pl.ds(wid * CPW, CPW)], dst_v)
        plsc.subcore_barrier()

        def body(j, carry):
            pltpu.async_copy(h_hbm.at[src_v.at[j]], rows_v, sem).wait()
            pltpu.sync_copy(rows_v, acc.at[dst_v.at[j]], add=True)
            return carry

        lax.fori_loop(0, CPW, body, None)
        plsc.subcore_barrier()

        pltpu.sync_copy(acc.at[pl.ds(sid * RPT_O, RPT_O)],
                        out_hbm.at[cid, pl.ds(sid * RPT_O, RPT_O)])

        @pl.when(sid == 0)
        def _():
            pltpu.sync_copy(acc.at[pl.ds(NS * RPT_O, TAIL_O)],
                            out_hbm.at[cid, pl.ds(NS * RPT_O, TAIL_O)])

    return k(h, src2d, dst2d, zeros)


def kernel(edge_index, features, weight, bias):
    ei = edge_index.astype(jnp.int32)
    pad = E_PAD - N_EDGES
    src = jnp.concatenate([ei[0], jnp.zeros((pad,), jnp.int32)])
    dst = jnp.concatenate([ei[1], jnp.full((pad,), N_NODES, jnp.int32)])
    src2d = src.reshape(NW * CPW, CHUNK)
    dst2d = dst.reshape(NW * CPW, CHUNK)
    h = _matmul(features, weight)
    zeros = jnp.zeros((ACC_ROWS, F), jnp.float32)
    p = _sc_scatter(h, src2d, dst2d, zeros)
    return _combine(p, bias.reshape(1, F))


# speedup vs baseline: 3.5357x; 1.0319x over previous
import functools

import jax
import jax.numpy as jnp
from jax import lax
from jax.experimental import pallas as pl
from jax.experimental.pallas import tpu as pltpu
from jax.experimental.pallas import tpu_sc as plsc

N_NODES = 10000
N_EDGES = 320000
F = 128

NC = 2
NS = 16
NW = NC * NS
CHUNK = 128
CPW = 80
HC = CPW // 2
E_PAD = NW * CPW * CHUNK
ACC_ROWS = 10240
RPT_Z = ACC_ROWS // NS
RPT_O = 624
TAIL_O = N_NODES - NS * RPT_O
ROW_BLK = N_NODES // 10


def _matmul_body(x_ref, w_ref, o_ref):
    o_ref[...] = jnp.dot(x_ref[...], w_ref[...],
                         preferred_element_type=jnp.float32)


def _matmul(x, w):
    return pl.pallas_call(
        _matmul_body,
        grid=(10,),
        in_specs=[
            pl.BlockSpec((ROW_BLK, F), lambda i: (i, 0)),
            pl.BlockSpec((F, F), lambda i: (0, 0)),
        ],
        out_specs=pl.BlockSpec((ROW_BLK, F), lambda i: (i, 0)),
        out_shape=jax.ShapeDtypeStruct((N_NODES, F), jnp.float32),
    )(x, w)


def _combine_body(p_ref, b_ref, o_ref):
    o_ref[...] = p_ref[0] + p_ref[1] + b_ref[...]


def _combine(p, b2d):
    return pl.pallas_call(
        _combine_body,
        grid=(10,),
        in_specs=[
            pl.BlockSpec((NC, ROW_BLK, F), lambda i: (0, i, 0)),
            pl.BlockSpec((1, F), lambda i: (0, 0)),
        ],
        out_specs=pl.BlockSpec((ROW_BLK, F), lambda i: (i, 0)),
        out_shape=jax.ShapeDtypeStruct((N_NODES, F), jnp.float32),
    )(p, b2d)


NBUF = 2
NGRP_H = HC // NBUF


def _sc_scatter(h, sd2d, zeros):
    mesh = plsc.VectorSubcoreMesh(core_axis_name="c", subcore_axis_name="s")

    @functools.partial(
        pl.kernel,
        mesh=mesh,
        out_type=jax.ShapeDtypeStruct((NC, N_NODES, F), jnp.float32),
        scratch_types=[
            pltpu.VMEM((HC, 2, CHUNK), jnp.int32),
            pltpu.VMEM((NBUF, CHUNK, F), jnp.float32),
            pltpu.VMEM_SHARED((ACC_ROWS, F), jnp.float32),
            [pltpu.SemaphoreType.DMA] * NBUF,
        ],
    )
    def k(h_hbm, sd_hbm, z_hbm, out_hbm, idx_v, rows_v, acc, gsem):
        cid = lax.axis_index("c")
        sid = lax.axis_index("s")
        wid = sid * NC + cid

        pltpu.sync_copy(z_hbm.at[pl.ds(sid * RPT_Z, RPT_Z)],
                        acc.at[pl.ds(sid * RPT_Z, RPT_Z)])
        plsc.subcore_barrier()

        def _gather(c, b):
            return pltpu.make_async_copy(h_hbm.at[idx_v.at[c, 0]],
                                         rows_v.at[b], gsem[b])

        for half in range(2):
            pltpu.sync_copy(sd_hbm.at[pl.ds(wid * CPW + half * HC, HC)],
                            idx_v)

            for b in range(NBUF):
                _gather(b, b).start()

            def body(g, carry):
                c0 = g * NBUF
                for b in range(NBUF):
                    _gather(c0 + b, b).wait()
                    pltpu.sync_copy(rows_v.at[b],
                                    acc.at[idx_v.at[c0 + b, 1]], add=True)

                    @pl.when(g + 1 < NGRP_H)
                    def _():
                        _gather(c0 + NBUF + b, b).start()

                return carry

            lax.fori_loop(0, NGRP_H, body, None)

        plsc.subcore_barrier()

        pltpu.sync_copy(acc.at[pl.ds(sid * RPT_O, RPT_O)],
                        out_hbm.at[cid, pl.ds(sid * RPT_O, RPT_O)])

        @pl.when(sid == 0)
        def _():
            pltpu.sync_copy(acc.at[pl.ds(NS * RPT_O, TAIL_O)],
                            out_hbm.at[cid, pl.ds(NS * RPT_O, TAIL_O)])

    return k(h, sd2d, zeros)


def kernel(edge_index, features, weight, bias):
    ei = edge_index.astype(jnp.int32)
    pad = E_PAD - N_EDGES
    src = jnp.concatenate([ei[0], jnp.zeros((pad,), jnp.int32)])
    dst = jnp.concatenate([ei[1], jnp.full((pad,), N_NODES, jnp.int32)])
    sd2d = jnp.stack([src.reshape(NW * CPW, CHUNK),
                      dst.reshape(NW * CPW, CHUNK)], axis=1)
    h = _matmul(features, weight)
    zeros = jnp.zeros((ACC_ROWS, F), jnp.float32)
    p = _sc_scatter(h, sd2d, zeros)
    return _combine(p, bias.reshape(1, F))


# speedup vs baseline: 11.8011x; 3.3377x over previous
import functools

import jax
import jax.numpy as jnp
from jax import lax
from jax.experimental import pallas as pl
from jax.experimental.pallas import tpu as pltpu
from jax.experimental.pallas import tpu_sc as plsc

N_NODES = 10000
N_EDGES = 320000
F = 128

NC = 2
NS = 16
NW = NC * NS
CHUNK = 64
CPW = 160
NQ = 4
QC = CPW // NQ
E_PAD = NW * CPW * CHUNK
ACC_ROWS = 10240
RPT_Z = ACC_ROWS // NS
RPT_O = 624
TAIL_O = N_NODES - NS * RPT_O
ROW_BLK = N_NODES // 10


def _matmul_body(x_ref, w_ref, o_ref):
    o_ref[...] = jnp.dot(x_ref[...], w_ref[...],
                         preferred_element_type=jnp.float32)


def _matmul(x, w):
    return pl.pallas_call(
        _matmul_body,
        grid=(10,),
        in_specs=[
            pl.BlockSpec((ROW_BLK, F), lambda i: (i, 0)),
            pl.BlockSpec((F, F), lambda i: (0, 0)),
        ],
        out_specs=pl.BlockSpec((ROW_BLK, F), lambda i: (i, 0)),
        out_shape=jax.ShapeDtypeStruct((N_NODES, F), jnp.float32),
    )(x, w)


def _combine_body(p_ref, b_ref, o_ref):
    o_ref[...] = p_ref[0] + p_ref[1] + b_ref[...]


def _combine(p, b2d):
    return pl.pallas_call(
        _combine_body,
        grid=(10,),
        in_specs=[
            pl.BlockSpec((NC, ROW_BLK, F), lambda i: (0, i, 0)),
            pl.BlockSpec((1, F), lambda i: (0, 0)),
        ],
        out_specs=pl.BlockSpec((ROW_BLK, F), lambda i: (i, 0)),
        out_shape=jax.ShapeDtypeStruct((N_NODES, F), jnp.float32),
    )(p, b2d)


NBUF = 4
NGRP_Q = QC // NBUF


def _sc_scatter(h, sd2d, zeros):
    mesh = plsc.VectorSubcoreMesh(core_axis_name="c", subcore_axis_name="s")

    @functools.partial(
        pl.kernel,
        mesh=mesh,
        out_type=jax.ShapeDtypeStruct((NC, N_NODES, F), jnp.float32),
        scratch_types=[
            pltpu.VMEM((QC, 2, CHUNK), jnp.int32),
            pltpu.VMEM((NBUF, CHUNK, F), jnp.float32),
            pltpu.VMEM_SHARED((ACC_ROWS, F), jnp.float32),
            [pltpu.SemaphoreType.DMA] * NBUF,
        ],
    )
    def k(h_hbm, sd_hbm, z_hbm, out_hbm, idx_v, rows_v, acc, gsem):
        cid = lax.axis_index("c")
        sid = lax.axis_index("s")
        wid = sid * NC + cid

        pltpu.sync_copy(z_hbm.at[pl.ds(sid * RPT_Z, RPT_Z)],
                        acc.at[pl.ds(sid * RPT_Z, RPT_Z)])
        plsc.subcore_barrier()

        def _gather(c, b):
            return pltpu.make_async_copy(h_hbm.at[idx_v.at[c, 0]],
                                         rows_v.at[b], gsem[b])

        for q in range(NQ):
            pltpu.sync_copy(sd_hbm.at[pl.ds(wid * CPW + q * QC, QC)],
                            idx_v)

            for b in range(NBUF):
                _gather(b, b).start()

            def body(g, carry):
                c0 = g * NBUF
                for b in range(NBUF):
                    _gather(c0 + b, b).wait()
                    pltpu.sync_copy(rows_v.at[b],
                                    acc.at[idx_v.at[c0 + b, 1]], add=True)

                    @pl.when(g + 1 < NGRP_Q)
                    def _():
                        _gather(c0 + NBUF + b, b).start()

                return carry

            lax.fori_loop(0, NGRP_Q, body, None)

        plsc.subcore_barrier()

        pltpu.sync_copy(acc.at[pl.ds(sid * RPT_O, RPT_O)],
                        out_hbm.at[cid, pl.ds(sid * RPT_O, RPT_O)])

        @pl.when(sid == 0)
        def _():
            pltpu.sync_copy(acc.at[pl.ds(NS * RPT_O, TAIL_O)],
                            out_hbm.at[cid, pl.ds(NS * RPT_O, TAIL_O)])

    return k(h, sd2d, zeros)


def kernel(edge_index, features, weight, bias):
    ei = edge_index.astype(jnp.int32)
    pad = E_PAD - N_EDGES
    src = jnp.concatenate([ei[0], jnp.arange(pad, dtype=jnp.int32) % N_NODES])
    dst = jnp.concatenate([ei[1], jnp.full((pad,), N_NODES, jnp.int32)])
    sd2d = jnp.stack([src.reshape(NW * CPW, CHUNK),
                      dst.reshape(NW * CPW, CHUNK)], axis=1)
    h = _matmul(features, weight)
    zeros = jnp.zeros((ACC_ROWS, F), jnp.float32)
    p = _sc_scatter(h, sd2d, zeros)
    return _combine(p, bias.reshape(1, F))
